# baseline (speedup 1.0000x reference)
import functools

import jax
import jax.numpy as jnp
from jax import lax
from jax.experimental import pallas as pl
from jax.experimental.pallas import tpu as pltpu
from jax.experimental.pallas import tpu_sc as plsc

F32 = jnp.float32
_NC, _NS = 2, 16
_NW = _NC * _NS
_CH = 128
_DP = 48
_EB = 256


def _pad2(x, dp):
    r = dp - x.shape[-1]
    return jnp.pad(x, [(0, 0)] * (x.ndim - 1) + [(0, r)]) if r else x


def _g_body(ef_ref, we1_ref, be1_ref, e42_ref, out_ref):
    g = jnp.dot(ef_ref[...], we1_ref[...], preferred_element_type=F32)
    out_ref[...] = jnp.maximum(g + be1_ref[...], 0.0) + e42_ref[...]


def _bmm_body(g_ref, h_ref, we2m_ref, out_ref, *, d, dp):
    hT = h_ref[...].T.astype(jnp.bfloat16)
    gT = g_ref[...].T
    rt = jnp.dot(we2m_ref[...], hT, preferred_element_type=F32)
    acc = rt[0:dp, :] * gT[0:1, :]
    for k in range(1, d + 1):
        acc = acc + rt[k * dp:(k + 1) * dp, :] * gT[k:k + 1, :]
    out_ref[...] = acc.T


def _in_body(x_ref, w_ref, b_ref, o_ref):
    o_ref[...] = jnp.maximum(
        jnp.dot(x_ref[...], w_ref[...], preferred_element_type=F32)
        + b_ref[...], 0.0)


def _upd_body(agg_ref, out_ref, wm1_ref, wm2_ref, cb_ref, bm_ref, new_ref):
    o = out_ref[...]
    conv = agg_ref[0] + agg_ref[1] + o + cb_ref[...]
    m = jnp.maximum(conv, 0.0)
    new_ref[...] = (jnp.dot(m, wm1_ref[...], preferred_element_type=F32)
                    + jnp.dot(o, wm2_ref[...], preferred_element_type=F32)
                    + bm_ref[...])


def _upd_final_body(agg_ref, out_ref, wm1_ref, wm2_ref, cb_ref, bm_ref,
                    init_ref, new_ref):
    o = out_ref[...]
    conv = agg_ref[0] + agg_ref[1] + o + cb_ref[...]
    m = jnp.maximum(conv, 0.0)
    new_ref[...] = (jnp.dot(m, wm1_ref[...], preferred_element_type=F32)
                    + jnp.dot(o, wm2_ref[...], preferred_element_type=F32)
                    + bm_ref[...] + init_ref[...])


def _make_gather(n, e, dp):
    epw = e // _NW
    nfull = epw // _CH
    mesh = plsc.VectorSubcoreMesh(core_axis_name="c", subcore_axis_name="s",
                                  num_cores=_NC, num_subcores=_NS)

    @functools.partial(
        pl.kernel,
        out_type=jax.ShapeDtypeStruct((e, dp), F32),
        mesh=mesh,
        compiler_params=pltpu.CompilerParams(use_tc_tiling_on_sc=False),
        scratch_types=[
            pltpu.VMEM((_CH,), jnp.int32),
            pltpu.VMEM((_CH, dp), F32),
            pltpu.SemaphoreType.DMA,
        ],
    )
    def gather_k(table_hbm, idx_hbm, out_hbm, idx_v, rows_v, sem):
        wid = lax.axis_index("s") * _NC + lax.axis_index("c")
        base = wid * epw

        def chunk(off):
            pltpu.sync_copy(idx_hbm.at[pl.ds(off, _CH)], idx_v)
            pltpu.async_copy(table_hbm.at[idx_v], rows_v, sem).wait()
            pltpu.sync_copy(rows_v, out_hbm.at[pl.ds(off, _CH)])

        def body(j, carry):
            chunk(base + j * _CH)
            return carry

        lax.fori_loop(0, nfull, body, 0)
        chunk(base + epw - _CH)

    return gather_k


def _make_scatter(n, e, dp):
    epw = e // _NW
    nfull = epw // _CH
    tail = epw - nfull * _CH
    npw = n // _NS
    mesh = plsc.VectorSubcoreMesh(core_axis_name="c", subcore_axis_name="s",
                                  num_cores=_NC, num_subcores=_NS)

    @functools.partial(
        pl.kernel,
        out_type=jax.ShapeDtypeStruct((_NC, n, dp), F32),
        mesh=mesh,
        compiler_params=pltpu.CompilerParams(use_tc_tiling_on_sc=False),
        scratch_types=[
            pltpu.VMEM((_CH,), jnp.int32),
            pltpu.VMEM((_CH, dp), F32),
            pltpu.VMEM((tail,), jnp.int32),
            pltpu.VMEM((tail, dp), F32),
            pltpu.VMEM_SHARED((n, dp), F32),
            pltpu.SemaphoreType.DMA,
        ],
    )
    def scatter_k(msg_hbm, dst_hbm, zero_hbm, out_hbm,
                  idx_v, rows_v, idx_t, rows_t, acc_s, sem):
        cid = lax.axis_index("c")
        sid = lax.axis_index("s")
        wid = sid * _NC + cid
        base = wid * epw
        pltpu.sync_copy(zero_hbm.at[pl.ds(sid * npw, npw)],
                        acc_s.at[pl.ds(sid * npw, npw)])
        plsc.subcore_barrier()

        def body(j, carry):
            off = base + j * _CH
            pltpu.sync_copy(dst_hbm.at[pl.ds(off, _CH)], idx_v)
            pltpu.sync_copy(msg_hbm.at[pl.ds(off, _CH)], rows_v)
            pltpu.async_copy(rows_v, acc_s.at[idx_v], sem, add=True).wait()
            return carry

        lax.fori_loop(0, nfull, body, 0)
        off = base + nfull * _CH
        pltpu.sync_copy(dst_hbm.at[pl.ds(off, tail)], idx_t)
        pltpu.sync_copy(msg_hbm.at[pl.ds(off, tail)], rows_t)
        pltpu.async_copy(rows_t, acc_s.at[idx_t], sem, add=True).wait()
        plsc.subcore_barrier()
        pltpu.sync_copy(acc_s.at[pl.ds(sid * npw, npw)],
                        out_hbm.at[cid, pl.ds(sid * npw, npw)])

    return scatter_k


def kernel(n_feat, edge_index, e_feat, W0, b0, We1, be1, We2, be2,
           conv_bias, Wm, bm):
    n, d = n_feat.shape
    e, de = e_feat.shape
    dp = _DP
    eb = _EB
    nbe = e // eb
    nbn = n // 10
    steps = 6
    src = edge_index[0]
    dst = edge_index[1]

    n_feat_p = _pad2(n_feat, dp)
    W0_p = jnp.pad(W0, ((0, dp - d), (0, dp - d)))
    b0_p = _pad2(b0[None, :], dp)
    We1p = _pad2(We1, dp)
    be1p = _pad2(be1[None, :], dp)
    e42 = (jnp.arange(dp) == d).astype(F32)[None, :]
    w2k = jnp.transpose(We2.reshape(d, d, d), (0, 2, 1))
    w2k = jnp.pad(w2k, ((0, 0), (0, dp - d), (0, dp - d)))
    b2k = jnp.pad(be2.reshape(d, d).T, ((0, dp - d), (0, dp - d)))[None]
    We2M = jnp.concatenate([w2k, b2k], 0).reshape((d + 1) * dp, dp)
    We2M = We2M.astype(jnp.bfloat16)
    cb = _pad2(conv_bias[None, :], dp)
    Wm1 = jnp.pad(Wm[:d], ((0, dp - d), (0, dp - d)))
    Wm2 = jnp.pad(Wm[d:], ((0, dp - d), (0, dp - d)))
    bmp = _pad2(bm[None, :], dp)
    zero_nd = jnp.zeros((n, dp), F32)

    gmat = pl.pallas_call(
        _g_body,
        grid=(nbe,),
        in_specs=[
            pl.BlockSpec((eb, de), lambda i: (i, 0)),
            pl.BlockSpec((de, dp), lambda i: (0, 0)),
            pl.BlockSpec((1, dp), lambda i: (0, 0)),
            pl.BlockSpec((1, dp), lambda i: (0, 0)),
        ],
        out_specs=pl.BlockSpec((eb, dp), lambda i: (i, 0)),
        out_shape=jax.ShapeDtypeStruct((e, dp), F32),
    )(e_feat, We1p, be1p, e42)

    out0 = pl.pallas_call(
        _in_body,
        grid=(n // nbn,),
        in_specs=[
            pl.BlockSpec((nbn, dp), lambda i: (i, 0)),
            pl.BlockSpec((dp, dp), lambda i: (0, 0)),
            pl.BlockSpec((1, dp), lambda i: (0, 0)),
        ],
        out_specs=pl.BlockSpec((nbn, dp), lambda i: (i, 0)),
        out_shape=jax.ShapeDtypeStruct((n, dp), F32),
    )(n_feat_p, W0_p, b0_p)

    gather_k = _make_gather(n, e, dp)
    scatter_k = _make_scatter(n, e, dp)

    bmm = pl.pallas_call(
        functools.partial(_bmm_body, d=d, dp=dp),
        grid=(nbe,),
        in_specs=[
            pl.BlockSpec((eb, dp), lambda i: (i, 0)),
            pl.BlockSpec((eb, dp), lambda i: (i, 0)),
            pl.BlockSpec(((d + 1) * dp, dp), lambda i: (0, 0)),
        ],
        out_specs=pl.BlockSpec((eb, dp), lambda i: (i, 0)),
        out_shape=jax.ShapeDtypeStruct((e, dp), F32),
    )

    upd_specs = [
        pl.BlockSpec((_NC, nbn, dp), lambda i: (0, i, 0)),
        pl.BlockSpec((nbn, dp), lambda i: (i, 0)),
        pl.BlockSpec((dp, dp), lambda i: (0, 0)),
        pl.BlockSpec((dp, dp), lambda i: (0, 0)),
        pl.BlockSpec((1, dp), lambda i: (0, 0)),
        pl.BlockSpec((1, dp), lambda i: (0, 0)),
    ]
    upd = pl.pallas_call(
        _upd_body,
        grid=(n // nbn,),
        in_specs=upd_specs,
        out_specs=pl.BlockSpec((nbn, dp), lambda i: (i, 0)),
        out_shape=jax.ShapeDtypeStruct((n, dp), F32),
    )
    upd_final = pl.pallas_call(
        _upd_final_body,
        grid=(n // nbn,),
        in_specs=upd_specs + [pl.BlockSpec((nbn, dp), lambda i: (i, 0))],
        out_specs=pl.BlockSpec((nbn, dp), lambda i: (i, 0)),
        out_shape=jax.ShapeDtypeStruct((n, dp), F32),
    )

    out = out0
    for t in range(steps):
        h_src = gather_k(out, src)
        msg = bmm(gmat, h_src, We2M)
        agg2 = scatter_k(msg, dst, zero_nd)
        if t < steps - 1:
            out = upd(agg2, out, Wm1, Wm2, cb, bmp)
        else:
            out = upd_final(agg2, out, Wm1, Wm2, cb, bmp, n_feat_p)
    return out[:, :d]


# speedup vs baseline: 1.1021x; 1.1021x over previous
import functools

import jax
import jax.numpy as jnp
from jax import lax
from jax.experimental import pallas as pl
from jax.experimental.pallas import tpu as pltpu
from jax.experimental.pallas import tpu_sc as plsc

F32 = jnp.float32
_NC, _NS = 2, 16
_NW = _NC * _NS
_CH = 128
_DP = 48
_EB = 256


def _pad2(x, dp):
    r = dp - x.shape[-1]
    return jnp.pad(x, [(0, 0)] * (x.ndim - 1) + [(0, r)]) if r else x


def _g_body(ef_ref, we1_ref, be1_ref, e42_ref, out_ref):
    g = jnp.dot(ef_ref[...], we1_ref[...], preferred_element_type=F32)
    out_ref[...] = jnp.maximum(g + be1_ref[...], 0.0) + e42_ref[...]


def _bmm_body(g_ref, h_ref, we2m_ref, out_ref, *, d, dp):
    hT = h_ref[...].T.astype(jnp.bfloat16)
    gT = g_ref[...].T

    def term(k):
        rtk = jnp.dot(we2m_ref[k * dp:(k + 1) * dp, :], hT,
                      preferred_element_type=F32)
        return rtk * gT[k:k + 1, :]

    acc0 = term(0)
    acc1 = term(1)
    for k in range(2, d + 1, 2):
        acc0 = acc0 + term(k)
        if k + 1 <= d:
            acc1 = acc1 + term(k + 1)
    out_ref[...] = (acc0 + acc1).T


def _in_body(x_ref, w_ref, b_ref, o_ref):
    o_ref[...] = jnp.maximum(
        jnp.dot(x_ref[...], w_ref[...], preferred_element_type=F32)
        + b_ref[...], 0.0)


def _upd_body(agg_ref, out_ref, wm1_ref, wm2_ref, cb_ref, bm_ref, new_ref):
    o = out_ref[...]
    conv = agg_ref[0] + agg_ref[1] + o + cb_ref[...]
    m = jnp.maximum(conv, 0.0)
    new_ref[...] = (jnp.dot(m, wm1_ref[...], preferred_element_type=F32)
                    + jnp.dot(o, wm2_ref[...], preferred_element_type=F32)
                    + bm_ref[...])


def _upd_final_body(agg_ref, out_ref, wm1_ref, wm2_ref, cb_ref, bm_ref,
                    init_ref, new_ref):
    o = out_ref[...]
    conv = agg_ref[0] + agg_ref[1] + o + cb_ref[...]
    m = jnp.maximum(conv, 0.0)
    new_ref[...] = (jnp.dot(m, wm1_ref[...], preferred_element_type=F32)
                    + jnp.dot(o, wm2_ref[...], preferred_element_type=F32)
                    + bm_ref[...] + init_ref[...])


def _make_gather(n, e, dp):
    nrow = e // _CH
    rpw = nrow // _NW
    extra = nrow - rpw * _NW
    mesh = plsc.VectorSubcoreMesh(core_axis_name="c", subcore_axis_name="s",
                                  num_cores=_NC, num_subcores=_NS)

    @functools.partial(
        pl.kernel,
        out_type=jax.ShapeDtypeStruct((e, dp), F32),
        mesh=mesh,
        compiler_params=pltpu.CompilerParams(use_tc_tiling_on_sc=False),
        scratch_types=[
            pltpu.VMEM((rpw + 1, _CH), jnp.int32),
            pltpu.VMEM((_CH, dp), F32),
            pltpu.VMEM((_CH, dp), F32),
            pltpu.SemaphoreType.DMA,
            pltpu.SemaphoreType.DMA,
            pltpu.SemaphoreType.DMA,
            pltpu.SemaphoreType.DMA,
        ],
    )
    def gather_k(table_hbm, idx_hbm, out_hbm, idxb, rows0, rows1,
                 g0, g1, w0, w1):
        wid = lax.axis_index("s") * _NC + lax.axis_index("c")
        rowbase = wid * rpw + jnp.minimum(wid, extra)
        has_extra = wid < extra

        pltpu.sync_copy(idx_hbm.at[pl.ds(rowbase, rpw)],
                        idxb.at[pl.ds(0, rpw)])

        @pl.when(has_extra)
        def _():
            pltpu.sync_copy(idx_hbm.at[pl.ds(rowbase + rpw, 1)],
                            idxb.at[pl.ds(rpw, 1)])

        def gat(j, rows, sem):
            return pltpu.async_copy(table_hbm.at[idxb.at[j]], rows, sem)

        def wrt(j, rows, sem):
            return pltpu.async_copy(
                rows, out_hbm.at[pl.ds((rowbase + j) * _CH, _CH)], sem)

        def wait_g(rows, sem):
            pltpu.make_async_copy(table_hbm.at[pl.ds(0, _CH)], rows,
                                  sem).wait()

        def wait_w(rows, sem):
            pltpu.make_async_copy(rows, out_hbm.at[pl.ds(0, _CH)],
                                  sem).wait()

        last = rpw - 1
        gat(0, rows0, g0)
        gat(1, rows1, g1)
        wait_g(rows0, g0)
        wrt(0, rows0, w0)

        def body(gi, carry):
            j1 = 2 * gi + 1
            wait_w(rows0, w0)
            gat(jnp.minimum(j1 + 1, last), rows0, g0)
            wait_g(rows1, g1)
            wrt(j1, rows1, w1)
            j2 = 2 * gi + 2
            wait_w(rows1, w1)
            gat(jnp.minimum(j2 + 1, last), rows1, g1)
            wait_g(rows0, g0)
            wrt(j2, rows0, w0)
            return carry

        lax.fori_loop(0, (rpw - 1) // 2, body, 0)
        wait_g(rows1, g1)
        wait_w(rows0, w0)

        @pl.when(has_extra)
        def _():
            gat(rpw, rows1, g1).wait()
            wrt(rpw, rows1, w1).wait()

    return gather_k


def _make_scatter(n, e, dp):
    nrow = e // _CH
    rpw = nrow // _NW
    extra = nrow - rpw * _NW
    npw = n // _NS
    mesh = plsc.VectorSubcoreMesh(core_axis_name="c", subcore_axis_name="s",
                                  num_cores=_NC, num_subcores=_NS)

    @functools.partial(
        pl.kernel,
        out_type=jax.ShapeDtypeStruct((_NC, n, dp), F32),
        mesh=mesh,
        compiler_params=pltpu.CompilerParams(use_tc_tiling_on_sc=False),
        scratch_types=[
            pltpu.VMEM((rpw + 1, _CH), jnp.int32),
            pltpu.VMEM((_CH, dp), F32),
            pltpu.VMEM((_CH, dp), F32),
            pltpu.VMEM_SHARED((n, dp), F32),
            pltpu.SemaphoreType.DMA,
            pltpu.SemaphoreType.DMA,
            pltpu.SemaphoreType.DMA,
        ],
    )
    def scatter_k(msg_hbm, dst_hbm, zero_hbm, out_hbm,
                  idxb, rows0, rows1, acc_s, l0, l1, s):
        cid = lax.axis_index("c")
        sid = lax.axis_index("s")
        wid = sid * _NC + cid
        rowbase = wid * rpw + jnp.minimum(wid, extra)
        has_extra = wid < extra

        pltpu.sync_copy(zero_hbm.at[pl.ds(sid * npw, npw)],
                        acc_s.at[pl.ds(sid * npw, npw)])
        pltpu.sync_copy(dst_hbm.at[pl.ds(rowbase, rpw)],
                        idxb.at[pl.ds(0, rpw)])

        @pl.when(has_extra)
        def _():
            pltpu.sync_copy(dst_hbm.at[pl.ds(rowbase + rpw, 1)],
                            idxb.at[pl.ds(rpw, 1)])

        plsc.subcore_barrier()

        def lod(j, rows, sem):
            return pltpu.async_copy(
                msg_hbm.at[pl.ds((rowbase + j) * _CH, _CH)], rows, sem)

        def wait_l(rows, sem):
            pltpu.make_async_copy(msg_hbm.at[pl.ds(0, _CH)], rows,
                                  sem).wait()

        def sca(j, rows):
            pltpu.async_copy(rows, acc_s.at[idxb.at[j]], s, add=True).wait()

        last = rpw - 1
        lod(0, rows0, l0)
        lod(1, rows1, l1)

        def body(gi, carry):
            j1 = 2 * gi
            wait_l(rows0, l0)
            sca(j1, rows0)
            lod(jnp.minimum(j1 + 2, last), rows0, l0)
            j2 = 2 * gi + 1
            wait_l(rows1, l1)
            sca(j2, rows1)
            lod(jnp.minimum(j2 + 2, last), rows1, l1)
            return carry

        lax.fori_loop(0, (rpw - 1) // 2, body, 0)
        wait_l(rows0, l0)
        sca(last, rows0)
        wait_l(rows1, l1)

        @pl.when(has_extra)
        def _():
            lod(rpw, rows1, l1).wait()
            sca(rpw, rows1)

        plsc.subcore_barrier()
        pltpu.sync_copy(acc_s.at[pl.ds(sid * npw, npw)],
                        out_hbm.at[cid, pl.ds(sid * npw, npw)])

    return scatter_k


def kernel(n_feat, edge_index, e_feat, W0, b0, We1, be1, We2, be2,
           conv_bias, Wm, bm):
    n, d = n_feat.shape
    e, de = e_feat.shape
    dp = _DP
    eb = _EB
    nbe = e // eb
    nbn = n // 10
    steps = 6
    src2 = edge_index[0].reshape(e // _CH, _CH)
    dst2 = edge_index[1].reshape(e // _CH, _CH)

    n_feat_p = _pad2(n_feat, dp)
    W0_p = jnp.pad(W0, ((0, dp - d), (0, dp - d)))
    b0_p = _pad2(b0[None, :], dp)
    We1p = _pad2(We1, dp)
    be1p = _pad2(be1[None, :], dp)
    e42 = (jnp.arange(dp) == d).astype(F32)[None, :]
    w2k = jnp.transpose(We2.reshape(d, d, d), (0, 2, 1))
    w2k = jnp.pad(w2k, ((0, 0), (0, dp - d), (0, dp - d)))
    b2k = jnp.pad(be2.reshape(d, d).T, ((0, dp - d), (0, dp - d)))[None]
    We2M = jnp.concatenate([w2k, b2k], 0).reshape((d + 1) * dp, dp)
    We2M = We2M.astype(jnp.bfloat16)
    cb = _pad2(conv_bias[None, :], dp)
    Wm1 = jnp.pad(Wm[:d], ((0, dp - d), (0, dp - d)))
    Wm2 = jnp.pad(Wm[d:], ((0, dp - d), (0, dp - d)))
    bmp = _pad2(bm[None, :], dp)
    zero_nd = jnp.zeros((n, dp), F32)

    gmat = pl.pallas_call(
        _g_body,
        grid=(nbe,),
        in_specs=[
            pl.BlockSpec((eb, de), lambda i: (i, 0)),
            pl.BlockSpec((de, dp), lambda i: (0, 0)),
            pl.BlockSpec((1, dp), lambda i: (0, 0)),
            pl.BlockSpec((1, dp), lambda i: (0, 0)),
        ],
        out_specs=pl.BlockSpec((eb, dp), lambda i: (i, 0)),
        out_shape=jax.ShapeDtypeStruct((e, dp), F32),
    )(e_feat, We1p, be1p, e42)

    out0 = pl.pallas_call(
        _in_body,
        grid=(n // nbn,),
        in_specs=[
            pl.BlockSpec((nbn, dp), lambda i: (i, 0)),
            pl.BlockSpec((dp, dp), lambda i: (0, 0)),
            pl.BlockSpec((1, dp), lambda i: (0, 0)),
        ],
        out_specs=pl.BlockSpec((nbn, dp), lambda i: (i, 0)),
        out_shape=jax.ShapeDtypeStruct((n, dp), F32),
    )(n_feat_p, W0_p, b0_p)

    gather_k = _make_gather(n, e, dp)
    scatter_k = _make_scatter(n, e, dp)

    bmm = pl.pallas_call(
        functools.partial(_bmm_body, d=d, dp=dp),
        grid=(nbe,),
        in_specs=[
            pl.BlockSpec((eb, dp), lambda i: (i, 0)),
            pl.BlockSpec((eb, dp), lambda i: (i, 0)),
            pl.BlockSpec(((d + 1) * dp, dp), lambda i: (0, 0)),
        ],
        out_specs=pl.BlockSpec((eb, dp), lambda i: (i, 0)),
        out_shape=jax.ShapeDtypeStruct((e, dp), F32),
    )

    upd_specs = [
        pl.BlockSpec((_NC, nbn, dp), lambda i: (0, i, 0)),
        pl.BlockSpec((nbn, dp), lambda i: (i, 0)),
        pl.BlockSpec((dp, dp), lambda i: (0, 0)),
        pl.BlockSpec((dp, dp), lambda i: (0, 0)),
        pl.BlockSpec((1, dp), lambda i: (0, 0)),
        pl.BlockSpec((1, dp), lambda i: (0, 0)),
    ]
    upd = pl.pallas_call(
        _upd_body,
        grid=(n // nbn,),
        in_specs=upd_specs,
        out_specs=pl.BlockSpec((nbn, dp), lambda i: (i, 0)),
        out_shape=jax.ShapeDtypeStruct((n, dp), F32),
    )
    upd_final = pl.pallas_call(
        _upd_final_body,
        grid=(n // nbn,),
        in_specs=upd_specs + [pl.BlockSpec((nbn, dp), lambda i: (i, 0))],
        out_specs=pl.BlockSpec((nbn, dp), lambda i: (i, 0)),
        out_shape=jax.ShapeDtypeStruct((n, dp), F32),
    )

    out = out0
    for t in range(steps):
        h_src = gather_k(out, src2)
        msg = bmm(gmat, h_src, We2M)
        agg2 = scatter_k(msg, dst2, zero_nd)
        if t < steps - 1:
            out = upd(agg2, out, Wm1, Wm2, cb, bmp)
        else:
            out = upd_final(agg2, out, Wm1, Wm2, cb, bmp, n_feat_p)
    return out[:, :d]


# speedup vs baseline: 1.4780x; 1.3410x over previous
import functools

import jax
import jax.numpy as jnp
from jax import lax
from jax.experimental import pallas as pl
from jax.experimental.pallas import tpu as pltpu
from jax.experimental.pallas import tpu_sc as plsc

F32 = jnp.float32
_NC, _NS = 2, 16
_NW = _NC * _NS
_CH = 128
_DP = 48
_EB = 512


def _pad2(x, dp):
    r = dp - x.shape[-1]
    return jnp.pad(x, [(0, 0)] * (x.ndim - 1) + [(0, r)]) if r else x


def _g_body(ef_ref, we1_ref, be1_ref, e42_ref, out_ref):
    g = jnp.dot(ef_ref[...], we1_ref[...], preferred_element_type=F32)
    out_ref[...] = jnp.maximum(g + be1_ref[...], 0.0) + e42_ref[...]


def _bmm_body(g_ref, h_ref, we2m_ref, out_ref, *, d, dp):
    hT = h_ref[...].T.astype(jnp.bfloat16)
    gT = g_ref[...].T

    def term(k):
        rtk = jnp.dot(we2m_ref[k * dp:(k + 1) * dp, :], hT,
                      preferred_element_type=F32)
        return rtk * gT[k:k + 1, :]

    acc0 = term(0)
    acc1 = term(1)
    for k in range(2, d + 1, 2):
        acc0 = acc0 + term(k)
        if k + 1 <= d:
            acc1 = acc1 + term(k + 1)
    out_ref[...] = (acc0 + acc1).T


def _in_body(x_ref, w_ref, b_ref, o_ref):
    o_ref[...] = jnp.maximum(
        jnp.dot(x_ref[...], w_ref[...], preferred_element_type=F32)
        + b_ref[...], 0.0)


def _upd_body(agg_ref, out_ref, wm1_ref, wm2_ref, cb_ref, bm_ref, new_ref):
    o = out_ref[...]
    conv = agg_ref[0] + agg_ref[1] + o + cb_ref[...]
    m = jnp.maximum(conv, 0.0)
    new_ref[...] = (jnp.dot(m, wm1_ref[...], preferred_element_type=F32)
                    + jnp.dot(o, wm2_ref[...], preferred_element_type=F32)
                    + bm_ref[...])


def _upd_final_body(agg_ref, out_ref, wm1_ref, wm2_ref, cb_ref, bm_ref,
                    init_ref, new_ref):
    o = out_ref[...]
    conv = agg_ref[0] + agg_ref[1] + o + cb_ref[...]
    m = jnp.maximum(conv, 0.0)
    new_ref[...] = (jnp.dot(m, wm1_ref[...], preferred_element_type=F32)
                    + jnp.dot(o, wm2_ref[...], preferred_element_type=F32)
                    + bm_ref[...] + init_ref[...])


def _make_gather(n, e, dp, ep):
    nrow = e // _CH
    rpw = nrow // _NW
    extra = nrow - rpw * _NW
    mesh = plsc.VectorSubcoreMesh(core_axis_name="c", subcore_axis_name="s",
                                  num_cores=_NC, num_subcores=_NS)

    @functools.partial(
        pl.kernel,
        out_type=jax.ShapeDtypeStruct((ep, dp), F32),
        mesh=mesh,
        compiler_params=pltpu.CompilerParams(use_tc_tiling_on_sc=False),
        scratch_types=[
            pltpu.VMEM((rpw + 1, _CH), jnp.int32),
            pltpu.VMEM((_CH, dp), F32),
            pltpu.VMEM((_CH, dp), F32),
            pltpu.SemaphoreType.DMA,
            pltpu.SemaphoreType.DMA,
            pltpu.SemaphoreType.DMA,
            pltpu.SemaphoreType.DMA,
        ],
    )
    def gather_k(table_hbm, idx_hbm, out_hbm, idxb, rows0, rows1,
                 g0, g1, w0, w1):
        wid = lax.axis_index("s") * _NC + lax.axis_index("c")
        rowbase = wid * rpw + jnp.minimum(wid, extra)
        has_extra = wid < extra

        pltpu.sync_copy(idx_hbm.at[pl.ds(rowbase, rpw)],
                        idxb.at[pl.ds(0, rpw)])

        @pl.when(has_extra)
        def _():
            pltpu.sync_copy(idx_hbm.at[pl.ds(rowbase + rpw, 1)],
                            idxb.at[pl.ds(rpw, 1)])

        def gat(j, rows, sem):
            return pltpu.async_copy(table_hbm.at[idxb.at[j]], rows, sem)

        def wrt(j, rows, sem):
            return pltpu.async_copy(
                rows, out_hbm.at[pl.ds((rowbase + j) * _CH, _CH)], sem)

        def wait_g(rows, sem):
            pltpu.make_async_copy(table_hbm.at[pl.ds(0, _CH)], rows,
                                  sem).wait()

        def wait_w(rows, sem):
            pltpu.make_async_copy(rows, out_hbm.at[pl.ds(0, _CH)],
                                  sem).wait()

        last = rpw - 1
        gat(0, rows0, g0)
        gat(1, rows1, g1)
        wait_g(rows0, g0)
        wrt(0, rows0, w0)

        def body(gi, carry):
            j1 = 2 * gi + 1
            wait_w(rows0, w0)
            gat(jnp.minimum(j1 + 1, last), rows0, g0)
            wait_g(rows1, g1)
            wrt(j1, rows1, w1)
            j2 = 2 * gi + 2
            wait_w(rows1, w1)
            gat(jnp.minimum(j2 + 1, last), rows1, g1)
            wait_g(rows0, g0)
            wrt(j2, rows0, w0)
            return carry

        lax.fori_loop(0, (rpw - 1) // 2, body, 0)
        wait_g(rows1, g1)
        wait_w(rows0, w0)

        @pl.when(has_extra)
        def _():
            gat(rpw, rows1, g1).wait()
            wrt(rpw, rows1, w1).wait()

    return gather_k


def _make_scatter(n, e, dp):
    nrow = e // _CH
    rpw = nrow // _NW
    extra = nrow - rpw * _NW
    npw = n // _NS
    mesh = plsc.VectorSubcoreMesh(core_axis_name="c", subcore_axis_name="s",
                                  num_cores=_NC, num_subcores=_NS)

    @functools.partial(
        pl.kernel,
        out_type=jax.ShapeDtypeStruct((_NC, n, dp), F32),
        mesh=mesh,
        compiler_params=pltpu.CompilerParams(use_tc_tiling_on_sc=False),
        scratch_types=[
            pltpu.VMEM((rpw + 1, _CH), jnp.int32),
            pltpu.VMEM((_CH, dp), F32),
            pltpu.VMEM((_CH, dp), F32),
            pltpu.VMEM_SHARED((n, dp), F32),
            pltpu.SemaphoreType.DMA,
            pltpu.SemaphoreType.DMA,
            pltpu.SemaphoreType.DMA,
        ],
    )
    def scatter_k(msg_hbm, dst_hbm, zero_hbm, out_hbm,
                  idxb, rows0, rows1, acc_s, l0, l1, s):
        cid = lax.axis_index("c")
        sid = lax.axis_index("s")
        wid = sid * _NC + cid
        rowbase = wid * rpw + jnp.minimum(wid, extra)
        has_extra = wid < extra

        pltpu.sync_copy(zero_hbm.at[pl.ds(sid * npw, npw)],
                        acc_s.at[pl.ds(sid * npw, npw)])
        pltpu.sync_copy(dst_hbm.at[pl.ds(rowbase, rpw)],
                        idxb.at[pl.ds(0, rpw)])

        @pl.when(has_extra)
        def _():
            pltpu.sync_copy(dst_hbm.at[pl.ds(rowbase + rpw, 1)],
                            idxb.at[pl.ds(rpw, 1)])

        plsc.subcore_barrier()

        def lod(j, rows, sem):
            return pltpu.async_copy(
                msg_hbm.at[pl.ds((rowbase + j) * _CH, _CH)], rows, sem)

        def wait_l(rows, sem):
            pltpu.make_async_copy(msg_hbm.at[pl.ds(0, _CH)], rows,
                                  sem).wait()

        def sca(j, rows):
            pltpu.async_copy(rows, acc_s.at[idxb.at[j]], s, add=True).wait()

        last = rpw - 1
        lod(0, rows0, l0)
        lod(1, rows1, l1)

        def body(gi, carry):
            j1 = 2 * gi
            wait_l(rows0, l0)
            sca(j1, rows0)
            lod(jnp.minimum(j1 + 2, last), rows0, l0)
            j2 = 2 * gi + 1
            wait_l(rows1, l1)
            sca(j2, rows1)
            lod(jnp.minimum(j2 + 2, last), rows1, l1)
            return carry

        lax.fori_loop(0, (rpw - 1) // 2, body, 0)
        wait_l(rows0, l0)
        sca(last, rows0)
        wait_l(rows1, l1)

        @pl.when(has_extra)
        def _():
            lod(rpw, rows1, l1).wait()
            sca(rpw, rows1)

        plsc.subcore_barrier()
        pltpu.sync_copy(acc_s.at[pl.ds(sid * npw, npw)],
                        out_hbm.at[cid, pl.ds(sid * npw, npw)])

    return scatter_k


def kernel(n_feat, edge_index, e_feat, W0, b0, We1, be1, We2, be2,
           conv_bias, Wm, bm):
    n, d = n_feat.shape
    e, de = e_feat.shape
    dp = _DP
    eb = _EB
    ep = -(-e // eb) * eb
    nbe = ep // eb
    nbn = n // 10
    steps = 6
    src2 = edge_index[0].reshape(e // _CH, _CH)
    dst2 = edge_index[1].reshape(e // _CH, _CH)

    n_feat_p = _pad2(n_feat, dp)
    W0_p = jnp.pad(W0, ((0, dp - d), (0, dp - d)))
    b0_p = _pad2(b0[None, :], dp)
    We1p = _pad2(We1, dp)
    be1p = _pad2(be1[None, :], dp)
    e42 = (jnp.arange(dp) == d).astype(F32)[None, :]
    w2k = jnp.transpose(We2.reshape(d, d, d), (0, 2, 1))
    w2k = jnp.pad(w2k, ((0, 0), (0, dp - d), (0, dp - d)))
    b2k = jnp.pad(be2.reshape(d, d).T, ((0, dp - d), (0, dp - d)))[None]
    We2M = jnp.concatenate([w2k, b2k], 0).reshape((d + 1) * dp, dp)
    We2M = We2M.astype(jnp.bfloat16)
    cb = _pad2(conv_bias[None, :], dp)
    Wm1 = jnp.pad(Wm[:d], ((0, dp - d), (0, dp - d)))
    Wm2 = jnp.pad(Wm[d:], ((0, dp - d), (0, dp - d)))
    bmp = _pad2(bm[None, :], dp)
    zero_nd = jnp.zeros((n, dp), F32)

    gmat = pl.pallas_call(
        _g_body,
        grid=(nbe,),
        in_specs=[
            pl.BlockSpec((eb, de), lambda i: (i, 0)),
            pl.BlockSpec((de, dp), lambda i: (0, 0)),
            pl.BlockSpec((1, dp), lambda i: (0, 0)),
            pl.BlockSpec((1, dp), lambda i: (0, 0)),
        ],
        out_specs=pl.BlockSpec((eb, dp), lambda i: (i, 0)),
        out_shape=jax.ShapeDtypeStruct((ep, dp), F32),
    )(jnp.pad(e_feat, ((0, ep - e), (0, 0))), We1p, be1p, e42)

    out0 = pl.pallas_call(
        _in_body,
        grid=(n // nbn,),
        in_specs=[
            pl.BlockSpec((nbn, dp), lambda i: (i, 0)),
            pl.BlockSpec((dp, dp), lambda i: (0, 0)),
            pl.BlockSpec((1, dp), lambda i: (0, 0)),
        ],
        out_specs=pl.BlockSpec((nbn, dp), lambda i: (i, 0)),
        out_shape=jax.ShapeDtypeStruct((n, dp), F32),
    )(n_feat_p, W0_p, b0_p)

    gather_k = _make_gather(n, e, dp, ep)
    scatter_k = _make_scatter(n, e, dp)

    bmm = pl.pallas_call(
        functools.partial(_bmm_body, d=d, dp=dp),
        grid=(nbe,),
        in_specs=[
            pl.BlockSpec((eb, dp), lambda i: (i, 0)),
            pl.BlockSpec((eb, dp), lambda i: (i, 0)),
            pl.BlockSpec(((d + 1) * dp, dp), lambda i: (0, 0)),
        ],
        out_specs=pl.BlockSpec((eb, dp), lambda i: (i, 0)),
        out_shape=jax.ShapeDtypeStruct((ep, dp), F32),
    )

    upd_specs = [
        pl.BlockSpec((_NC, nbn, dp), lambda i: (0, i, 0)),
        pl.BlockSpec((nbn, dp), lambda i: (i, 0)),
        pl.BlockSpec((dp, dp), lambda i: (0, 0)),
        pl.BlockSpec((dp, dp), lambda i: (0, 0)),
        pl.BlockSpec((1, dp), lambda i: (0, 0)),
        pl.BlockSpec((1, dp), lambda i: (0, 0)),
    ]
    upd = pl.pallas_call(
        _upd_body,
        grid=(n // nbn,),
        in_specs=upd_specs,
        out_specs=pl.BlockSpec((nbn, dp), lambda i: (i, 0)),
        out_shape=jax.ShapeDtypeStruct((n, dp), F32),
    )
    upd_final = pl.pallas_call(
        _upd_final_body,
        grid=(n // nbn,),
        in_specs=upd_specs + [pl.BlockSpec((nbn, dp), lambda i: (i, 0))],
        out_specs=pl.BlockSpec((nbn, dp), lambda i: (i, 0)),
        out_shape=jax.ShapeDtypeStruct((n, dp), F32),
    )

    out = out0
    for t in range(steps):
        h_src = gather_k(out, src2)
        msg = bmm(gmat, h_src, We2M)
        agg2 = scatter_k(msg, dst2, zero_nd)
        if t < steps - 1:
            out = upd(agg2, out, Wm1, Wm2, cb, bmp)
        else:
            out = upd_final(agg2, out, Wm1, Wm2, cb, bmp, n_feat_p)
    return out[:, :d]


# speedup vs baseline: 1.7653x; 1.1944x over previous
import functools

import jax
import jax.numpy as jnp
from jax import lax
from jax.experimental import pallas as pl
from jax.experimental.pallas import tpu as pltpu
from jax.experimental.pallas import tpu_sc as plsc

F32 = jnp.float32
_NC, _NS = 2, 16
_NW = _NC * _NS
_CH = 128
_DP = 48
_EB = 1024


def _pad2(x, dp):
    r = dp - x.shape[-1]
    return jnp.pad(x, [(0, 0)] * (x.ndim - 1) + [(0, r)]) if r else x


def _g_body(ef_ref, we1_ref, be1_ref, e42_ref, out_ref):
    g = jnp.dot(ef_ref[...], we1_ref[...], preferred_element_type=F32)
    out_ref[...] = jnp.maximum(g + be1_ref[...], 0.0) + e42_ref[...]


def _bmm_body(g_ref, h_ref, we2m_ref, out_ref, *, d, dp):
    hT = h_ref[...].T.astype(jnp.bfloat16)
    gT = g_ref[...].T

    def term(k):
        rtk = jnp.dot(we2m_ref[k * dp:(k + 1) * dp, :], hT,
                      preferred_element_type=F32)
        return rtk * gT[k:k + 1, :]

    acc0 = term(0)
    acc1 = term(1)
    for k in range(2, d + 1, 2):
        acc0 = acc0 + term(k)
        if k + 1 <= d:
            acc1 = acc1 + term(k + 1)
    out_ref[...] = (acc0 + acc1).T


def _in_body(x_ref, w_ref, b_ref, o_ref):
    o_ref[...] = jnp.maximum(
        jnp.dot(x_ref[...], w_ref[...], preferred_element_type=F32)
        + b_ref[...], 0.0)


def _upd_body(agg_ref, out_ref, wm1_ref, wm2_ref, cb_ref, bm_ref, new_ref):
    o = out_ref[...]
    conv = agg_ref[0] + agg_ref[1] + o + cb_ref[...]
    m = jnp.maximum(conv, 0.0)
    new_ref[...] = (jnp.dot(m, wm1_ref[...], preferred_element_type=F32)
                    + jnp.dot(o, wm2_ref[...], preferred_element_type=F32)
                    + bm_ref[...])


def _upd_final_body(agg_ref, out_ref, wm1_ref, wm2_ref, cb_ref, bm_ref,
                    init_ref, new_ref):
    o = out_ref[...]
    conv = agg_ref[0] + agg_ref[1] + o + cb_ref[...]
    m = jnp.maximum(conv, 0.0)
    new_ref[...] = (jnp.dot(m, wm1_ref[...], preferred_element_type=F32)
                    + jnp.dot(o, wm2_ref[...], preferred_element_type=F32)
                    + bm_ref[...] + init_ref[...])


def _make_gather(n, e, dp, ep):
    nrow = e // _CH
    rpw = nrow // _NW
    extra = nrow - rpw * _NW
    mesh = plsc.VectorSubcoreMesh(core_axis_name="c", subcore_axis_name="s",
                                  num_cores=_NC, num_subcores=_NS)

    @functools.partial(
        pl.kernel,
        out_type=jax.ShapeDtypeStruct((ep, dp), F32),
        mesh=mesh,
        compiler_params=pltpu.CompilerParams(use_tc_tiling_on_sc=False),
        scratch_types=[
            pltpu.VMEM((rpw + 1, _CH), jnp.int32),
            pltpu.VMEM((_CH, dp), F32),
            pltpu.VMEM((_CH, dp), F32),
            pltpu.SemaphoreType.DMA,
            pltpu.SemaphoreType.DMA,
            pltpu.SemaphoreType.DMA,
            pltpu.SemaphoreType.DMA,
        ],
    )
    def gather_k(table_hbm, idx_hbm, out_hbm, idxb, rows0, rows1,
                 g0, g1, w0, w1):
        wid = lax.axis_index("s") * _NC + lax.axis_index("c")
        rowbase = wid * rpw + jnp.minimum(wid, extra)
        has_extra = wid < extra

        pltpu.sync_copy(idx_hbm.at[pl.ds(rowbase, rpw)],
                        idxb.at[pl.ds(0, rpw)])

        @pl.when(has_extra)
        def _():
            pltpu.sync_copy(idx_hbm.at[pl.ds(rowbase + rpw, 1)],
                            idxb.at[pl.ds(rpw, 1)])

        def gat(j, rows, sem):
            return pltpu.async_copy(table_hbm.at[idxb.at[j]], rows, sem)

        def wrt(j, rows, sem):
            return pltpu.async_copy(
                rows, out_hbm.at[pl.ds((rowbase + j) * _CH, _CH)], sem)

        def wait_g(rows, sem):
            pltpu.make_async_copy(table_hbm.at[pl.ds(0, _CH)], rows,
                                  sem).wait()

        def wait_w(rows, sem):
            pltpu.make_async_copy(rows, out_hbm.at[pl.ds(0, _CH)],
                                  sem).wait()

        last = rpw - 1
        gat(0, rows0, g0)
        gat(1, rows1, g1)
        wait_g(rows0, g0)
        wrt(0, rows0, w0)

        def body(gi, carry):
            j1 = 2 * gi + 1
            wait_w(rows0, w0)
            gat(jnp.minimum(j1 + 1, last), rows0, g0)
            wait_g(rows1, g1)
            wrt(j1, rows1, w1)
            j2 = 2 * gi + 2
            wait_w(rows1, w1)
            gat(jnp.minimum(j2 + 1, last), rows1, g1)
            wait_g(rows0, g0)
            wrt(j2, rows0, w0)
            return carry

        lax.fori_loop(0, (rpw - 1) // 2, body, 0)
        wait_g(rows1, g1)
        wait_w(rows0, w0)

        @pl.when(has_extra)
        def _():
            gat(rpw, rows1, g1).wait()
            wrt(rpw, rows1, w1).wait()

    return gather_k


def _make_scatter(n, e, dp):
    nrow = e // _CH
    rpw = nrow // _NW
    extra = nrow - rpw * _NW
    npw = n // _NS
    mesh = plsc.VectorSubcoreMesh(core_axis_name="c", subcore_axis_name="s",
                                  num_cores=_NC, num_subcores=_NS)

    @functools.partial(
        pl.kernel,
        out_type=jax.ShapeDtypeStruct((_NC, n, dp), F32),
        mesh=mesh,
        compiler_params=pltpu.CompilerParams(use_tc_tiling_on_sc=False),
        scratch_types=[
            pltpu.VMEM((rpw + 1, _CH), jnp.int32),
            pltpu.VMEM((_CH, dp), F32),
            pltpu.VMEM((_CH, dp), F32),
            pltpu.VMEM_SHARED((n, dp), F32),
            pltpu.SemaphoreType.DMA,
            pltpu.SemaphoreType.DMA,
            pltpu.SemaphoreType.DMA,
        ],
    )
    def scatter_k(msg_hbm, dst_hbm, zero_hbm, out_hbm,
                  idxb, rows0, rows1, acc_s, l0, l1, s):
        cid = lax.axis_index("c")
        sid = lax.axis_index("s")
        wid = sid * _NC + cid
        rowbase = wid * rpw + jnp.minimum(wid, extra)
        has_extra = wid < extra

        pltpu.sync_copy(zero_hbm.at[pl.ds(sid * npw, npw)],
                        acc_s.at[pl.ds(sid * npw, npw)])
        pltpu.sync_copy(dst_hbm.at[pl.ds(rowbase, rpw)],
                        idxb.at[pl.ds(0, rpw)])

        @pl.when(has_extra)
        def _():
            pltpu.sync_copy(dst_hbm.at[pl.ds(rowbase + rpw, 1)],
                            idxb.at[pl.ds(rpw, 1)])

        plsc.subcore_barrier()

        def lod(j, rows, sem):
            return pltpu.async_copy(
                msg_hbm.at[pl.ds((rowbase + j) * _CH, _CH)], rows, sem)

        def wait_l(rows, sem):
            pltpu.make_async_copy(msg_hbm.at[pl.ds(0, _CH)], rows,
                                  sem).wait()

        def sca(j, rows):
            pltpu.async_copy(rows, acc_s.at[idxb.at[j]], s, add=True).wait()

        last = rpw - 1
        lod(0, rows0, l0)
        lod(1, rows1, l1)

        def body(gi, carry):
            j1 = 2 * gi
            wait_l(rows0, l0)
            sca(j1, rows0)
            lod(jnp.minimum(j1 + 2, last), rows0, l0)
            j2 = 2 * gi + 1
            wait_l(rows1, l1)
            sca(j2, rows1)
            lod(jnp.minimum(j2 + 2, last), rows1, l1)
            return carry

        lax.fori_loop(0, (rpw - 1) // 2, body, 0)
        wait_l(rows0, l0)
        sca(last, rows0)
        wait_l(rows1, l1)

        @pl.when(has_extra)
        def _():
            lod(rpw, rows1, l1).wait()
            sca(rpw, rows1)

        plsc.subcore_barrier()
        pltpu.sync_copy(acc_s.at[pl.ds(sid * npw, npw)],
                        out_hbm.at[cid, pl.ds(sid * npw, npw)])

    return scatter_k


def kernel(n_feat, edge_index, e_feat, W0, b0, We1, be1, We2, be2,
           conv_bias, Wm, bm):
    n, d = n_feat.shape
    e, de = e_feat.shape
    dp = _DP
    eb = _EB
    ep = -(-e // eb) * eb
    nbe = ep // eb
    nbn = n // 10
    steps = 6
    src2 = edge_index[0].reshape(e // _CH, _CH)
    dst2 = edge_index[1].reshape(e // _CH, _CH)

    n_feat_p = _pad2(n_feat, dp)
    W0_p = jnp.pad(W0, ((0, dp - d), (0, dp - d)))
    b0_p = _pad2(b0[None, :], dp)
    We1p = _pad2(We1, dp)
    be1p = _pad2(be1[None, :], dp)
    e42 = (jnp.arange(dp) == d).astype(F32)[None, :]
    w2k = jnp.transpose(We2.reshape(d, d, d), (0, 2, 1))
    w2k = jnp.pad(w2k, ((0, 0), (0, dp - d), (0, dp - d)))
    b2k = jnp.pad(be2.reshape(d, d).T, ((0, dp - d), (0, dp - d)))[None]
    We2M = jnp.concatenate([w2k, b2k], 0).reshape((d + 1) * dp, dp)
    We2M = We2M.astype(jnp.bfloat16)
    cb = _pad2(conv_bias[None, :], dp)
    Wm1 = jnp.pad(Wm[:d], ((0, dp - d), (0, dp - d)))
    Wm2 = jnp.pad(Wm[d:], ((0, dp - d), (0, dp - d)))
    bmp = _pad2(bm[None, :], dp)
    zero_nd = jnp.zeros((n, dp), F32)

    gmat = pl.pallas_call(
        _g_body,
        grid=(nbe,),
        in_specs=[
            pl.BlockSpec((eb, de), lambda i: (i, 0)),
            pl.BlockSpec((de, dp), lambda i: (0, 0)),
            pl.BlockSpec((1, dp), lambda i: (0, 0)),
            pl.BlockSpec((1, dp), lambda i: (0, 0)),
        ],
        out_specs=pl.BlockSpec((eb, dp), lambda i: (i, 0)),
        out_shape=jax.ShapeDtypeStruct((ep, dp), F32),
    )(jnp.pad(e_feat, ((0, ep - e), (0, 0))), We1p, be1p, e42)

    out0 = pl.pallas_call(
        _in_body,
        grid=(n // nbn,),
        in_specs=[
            pl.BlockSpec((nbn, dp), lambda i: (i, 0)),
            pl.BlockSpec((dp, dp), lambda i: (0, 0)),
            pl.BlockSpec((1, dp), lambda i: (0, 0)),
        ],
        out_specs=pl.BlockSpec((nbn, dp), lambda i: (i, 0)),
        out_shape=jax.ShapeDtypeStruct((n, dp), F32),
    )(n_feat_p, W0_p, b0_p)

    gather_k = _make_gather(n, e, dp, ep)
    scatter_k = _make_scatter(n, e, dp)

    bmm = pl.pallas_call(
        functools.partial(_bmm_body, d=d, dp=dp),
        grid=(nbe,),
        in_specs=[
            pl.BlockSpec((eb, dp), lambda i: (i, 0)),
            pl.BlockSpec((eb, dp), lambda i: (i, 0)),
            pl.BlockSpec(((d + 1) * dp, dp), lambda i: (0, 0)),
        ],
        out_specs=pl.BlockSpec((eb, dp), lambda i: (i, 0)),
        out_shape=jax.ShapeDtypeStruct((ep, dp), F32),
    )

    upd_specs = [
        pl.BlockSpec((_NC, nbn, dp), lambda i: (0, i, 0)),
        pl.BlockSpec((nbn, dp), lambda i: (i, 0)),
        pl.BlockSpec((dp, dp), lambda i: (0, 0)),
        pl.BlockSpec((dp, dp), lambda i: (0, 0)),
        pl.BlockSpec((1, dp), lambda i: (0, 0)),
        pl.BlockSpec((1, dp), lambda i: (0, 0)),
    ]
    upd = pl.pallas_call(
        _upd_body,
        grid=(n // nbn,),
        in_specs=upd_specs,
        out_specs=pl.BlockSpec((nbn, dp), lambda i: (i, 0)),
        out_shape=jax.ShapeDtypeStruct((n, dp), F32),
    )
    upd_final = pl.pallas_call(
        _upd_final_body,
        grid=(n // nbn,),
        in_specs=upd_specs + [pl.BlockSpec((nbn, dp), lambda i: (i, 0))],
        out_specs=pl.BlockSpec((nbn, dp), lambda i: (i, 0)),
        out_shape=jax.ShapeDtypeStruct((n, dp), F32),
    )

    out = out0
    for t in range(steps):
        h_src = gather_k(out, src2)
        msg = bmm(gmat, h_src, We2M)
        agg2 = scatter_k(msg, dst2, zero_nd)
        if t < steps - 1:
            out = upd(agg2, out, Wm1, Wm2, cb, bmp)
        else:
            out = upd_final(agg2, out, Wm1, Wm2, cb, bmp, n_feat_p)
    return out[:, :d]


# speedup vs baseline: 1.8596x; 1.0534x over previous
import functools

import jax
import jax.numpy as jnp
from jax import lax
from jax.experimental import pallas as pl
from jax.experimental.pallas import tpu as pltpu
from jax.experimental.pallas import tpu_sc as plsc

F32 = jnp.float32
_NC, _NS = 2, 16
_NW = _NC * _NS
_CH = 128
_DP = 48
_EB = 2048


def _pad2(x, dp):
    r = dp - x.shape[-1]
    return jnp.pad(x, [(0, 0)] * (x.ndim - 1) + [(0, r)]) if r else x


def _g_body(ef_ref, we1_ref, be1_ref, e42_ref, out_ref):
    g = jnp.dot(ef_ref[...], we1_ref[...], preferred_element_type=F32)
    out_ref[...] = jnp.maximum(g + be1_ref[...], 0.0) + e42_ref[...]


def _bmm_body(g_ref, h_ref, we2m_ref, out_ref, *, d, dp):
    hT = h_ref[...].T.astype(jnp.bfloat16)
    gT = g_ref[...].T

    def term(k):
        rtk = jnp.dot(we2m_ref[k * dp:(k + 1) * dp, :], hT,
                      preferred_element_type=F32)
        return rtk * gT[k:k + 1, :]

    acc0 = term(0)
    acc1 = term(1)
    for k in range(2, d + 1, 2):
        acc0 = acc0 + term(k)
        if k + 1 <= d:
            acc1 = acc1 + term(k + 1)
    out_ref[...] = (acc0 + acc1).T


def _in_body(x_ref, w_ref, b_ref, o_ref):
    o_ref[...] = jnp.maximum(
        jnp.dot(x_ref[...], w_ref[...], preferred_element_type=F32)
        + b_ref[...], 0.0)


def _upd_body(agg_ref, out_ref, wm1_ref, wm2_ref, cb_ref, bm_ref, new_ref):
    o = out_ref[...]
    conv = agg_ref[0] + agg_ref[1] + o + cb_ref[...]
    m = jnp.maximum(conv, 0.0)
    new_ref[...] = (jnp.dot(m, wm1_ref[...], preferred_element_type=F32)
                    + jnp.dot(o, wm2_ref[...], preferred_element_type=F32)
                    + bm_ref[...])


def _upd_final_body(agg_ref, out_ref, wm1_ref, wm2_ref, cb_ref, bm_ref,
                    init_ref, new_ref):
    o = out_ref[...]
    conv = agg_ref[0] + agg_ref[1] + o + cb_ref[...]
    m = jnp.maximum(conv, 0.0)
    new_ref[...] = (jnp.dot(m, wm1_ref[...], preferred_element_type=F32)
                    + jnp.dot(o, wm2_ref[...], preferred_element_type=F32)
                    + bm_ref[...] + init_ref[...])


def _make_gather(n, e, dp, ep):
    nrow = e // _CH
    rpw = nrow // _NW
    extra = nrow - rpw * _NW
    mesh = plsc.VectorSubcoreMesh(core_axis_name="c", subcore_axis_name="s",
                                  num_cores=_NC, num_subcores=_NS)

    @functools.partial(
        pl.kernel,
        out_type=jax.ShapeDtypeStruct((ep, dp), F32),
        mesh=mesh,
        compiler_params=pltpu.CompilerParams(use_tc_tiling_on_sc=False),
        scratch_types=[
            pltpu.VMEM((rpw + 1, _CH), jnp.int32),
            pltpu.VMEM((_CH, dp), F32),
            pltpu.VMEM((_CH, dp), F32),
            pltpu.SemaphoreType.DMA,
            pltpu.SemaphoreType.DMA,
            pltpu.SemaphoreType.DMA,
            pltpu.SemaphoreType.DMA,
        ],
    )
    def gather_k(table_hbm, idx_hbm, out_hbm, idxb, rows0, rows1,
                 g0, g1, w0, w1):
        wid = lax.axis_index("s") * _NC + lax.axis_index("c")
        rowbase = wid * rpw + jnp.minimum(wid, extra)
        has_extra = wid < extra

        pltpu.sync_copy(idx_hbm.at[pl.ds(rowbase, rpw)],
                        idxb.at[pl.ds(0, rpw)])

        @pl.when(has_extra)
        def _():
            pltpu.sync_copy(idx_hbm.at[pl.ds(rowbase + rpw, 1)],
                            idxb.at[pl.ds(rpw, 1)])

        def gat(j, rows, sem):
            return pltpu.async_copy(table_hbm.at[idxb.at[j]], rows, sem)

        def wrt(j, rows, sem):
            return pltpu.async_copy(
                rows, out_hbm.at[pl.ds((rowbase + j) * _CH, _CH)], sem)

        def wait_g(rows, sem):
            pltpu.make_async_copy(table_hbm.at[pl.ds(0, _CH)], rows,
                                  sem).wait()

        def wait_w(rows, sem):
            pltpu.make_async_copy(rows, out_hbm.at[pl.ds(0, _CH)],
                                  sem).wait()

        last = rpw - 1
        gat(0, rows0, g0)
        gat(1, rows1, g1)
        wait_g(rows0, g0)
        wrt(0, rows0, w0)

        def body(gi, carry):
            j1 = 2 * gi + 1
            wait_w(rows0, w0)
            gat(jnp.minimum(j1 + 1, last), rows0, g0)
            wait_g(rows1, g1)
            wrt(j1, rows1, w1)
            j2 = 2 * gi + 2
            wait_w(rows1, w1)
            gat(jnp.minimum(j2 + 1, last), rows1, g1)
            wait_g(rows0, g0)
            wrt(j2, rows0, w0)
            return carry

        lax.fori_loop(0, (rpw - 1) // 2, body, 0)
        wait_g(rows1, g1)
        wait_w(rows0, w0)

        @pl.when(has_extra)
        def _():
            gat(rpw, rows1, g1).wait()
            wrt(rpw, rows1, w1).wait()

    return gather_k


def _make_scatter(n, e, dp):
    nrow = e // _CH
    rpw = nrow // _NW
    extra = nrow - rpw * _NW
    npw = n // _NS
    mesh = plsc.VectorSubcoreMesh(core_axis_name="c", subcore_axis_name="s",
                                  num_cores=_NC, num_subcores=_NS)

    @functools.partial(
        pl.kernel,
        out_type=jax.ShapeDtypeStruct((_NC, n, dp), F32),
        mesh=mesh,
        compiler_params=pltpu.CompilerParams(use_tc_tiling_on_sc=False),
        scratch_types=[
            pltpu.VMEM((rpw + 1, _CH), jnp.int32),
            pltpu.VMEM((_CH, dp), F32),
            pltpu.VMEM((_CH, dp), F32),
            pltpu.VMEM_SHARED((n, dp), F32),
            pltpu.SemaphoreType.DMA,
            pltpu.SemaphoreType.DMA,
            pltpu.SemaphoreType.DMA,
        ],
    )
    def scatter_k(msg_hbm, dst_hbm, zero_hbm, out_hbm,
                  idxb, rows0, rows1, acc_s, l0, l1, s):
        cid = lax.axis_index("c")
        sid = lax.axis_index("s")
        wid = sid * _NC + cid
        rowbase = wid * rpw + jnp.minimum(wid, extra)
        has_extra = wid < extra

        pltpu.sync_copy(zero_hbm.at[pl.ds(sid * npw, npw)],
                        acc_s.at[pl.ds(sid * npw, npw)])
        pltpu.sync_copy(dst_hbm.at[pl.ds(rowbase, rpw)],
                        idxb.at[pl.ds(0, rpw)])

        @pl.when(has_extra)
        def _():
            pltpu.sync_copy(dst_hbm.at[pl.ds(rowbase + rpw, 1)],
                            idxb.at[pl.ds(rpw, 1)])

        plsc.subcore_barrier()

        def lod(j, rows, sem):
            return pltpu.async_copy(
                msg_hbm.at[pl.ds((rowbase + j) * _CH, _CH)], rows, sem)

        def wait_l(rows, sem):
            pltpu.make_async_copy(msg_hbm.at[pl.ds(0, _CH)], rows,
                                  sem).wait()

        def sca(j, rows):
            pltpu.async_copy(rows, acc_s.at[idxb.at[j]], s, add=True).wait()

        last = rpw - 1
        lod(0, rows0, l0)
        lod(1, rows1, l1)

        def body(gi, carry):
            j1 = 2 * gi
            wait_l(rows0, l0)
            sca(j1, rows0)
            lod(jnp.minimum(j1 + 2, last), rows0, l0)
            j2 = 2 * gi + 1
            wait_l(rows1, l1)
            sca(j2, rows1)
            lod(jnp.minimum(j2 + 2, last), rows1, l1)
            return carry

        lax.fori_loop(0, (rpw - 1) // 2, body, 0)
        wait_l(rows0, l0)
        sca(last, rows0)
        wait_l(rows1, l1)

        @pl.when(has_extra)
        def _():
            lod(rpw, rows1, l1).wait()
            sca(rpw, rows1)

        plsc.subcore_barrier()
        pltpu.sync_copy(acc_s.at[pl.ds(sid * npw, npw)],
                        out_hbm.at[cid, pl.ds(sid * npw, npw)])

    return scatter_k


def kernel(n_feat, edge_index, e_feat, W0, b0, We1, be1, We2, be2,
           conv_bias, Wm, bm):
    n, d = n_feat.shape
    e, de = e_feat.shape
    dp = _DP
    eb = _EB
    ep = -(-e // eb) * eb
    nbe = ep // eb
    nbn = n // 10
    steps = 6
    src2 = edge_index[0].reshape(e // _CH, _CH)
    dst2 = edge_index[1].reshape(e // _CH, _CH)

    n_feat_p = _pad2(n_feat, dp)
    W0_p = jnp.pad(W0, ((0, dp - d), (0, dp - d)))
    b0_p = _pad2(b0[None, :], dp)
    We1p = _pad2(We1, dp)
    be1p = _pad2(be1[None, :], dp)
    e42 = (jnp.arange(dp) == d).astype(F32)[None, :]
    w2k = jnp.transpose(We2.reshape(d, d, d), (0, 2, 1))
    w2k = jnp.pad(w2k, ((0, 0), (0, dp - d), (0, dp - d)))
    b2k = jnp.pad(be2.reshape(d, d).T, ((0, dp - d), (0, dp - d)))[None]
    We2M = jnp.concatenate([w2k, b2k], 0).reshape((d + 1) * dp, dp)
    We2M = We2M.astype(jnp.bfloat16)
    cb = _pad2(conv_bias[None, :], dp)
    Wm1 = jnp.pad(Wm[:d], ((0, dp - d), (0, dp - d)))
    Wm2 = jnp.pad(Wm[d:], ((0, dp - d), (0, dp - d)))
    bmp = _pad2(bm[None, :], dp)
    zero_nd = jnp.zeros((n, dp), F32)

    gmat = pl.pallas_call(
        _g_body,
        grid=(nbe,),
        in_specs=[
            pl.BlockSpec((eb, de), lambda i: (i, 0)),
            pl.BlockSpec((de, dp), lambda i: (0, 0)),
            pl.BlockSpec((1, dp), lambda i: (0, 0)),
            pl.BlockSpec((1, dp), lambda i: (0, 0)),
        ],
        out_specs=pl.BlockSpec((eb, dp), lambda i: (i, 0)),
        out_shape=jax.ShapeDtypeStruct((ep, dp), F32),
    )(jnp.pad(e_feat, ((0, ep - e), (0, 0))), We1p, be1p, e42)

    out0 = pl.pallas_call(
        _in_body,
        grid=(n // nbn,),
        in_specs=[
            pl.BlockSpec((nbn, dp), lambda i: (i, 0)),
            pl.BlockSpec((dp, dp), lambda i: (0, 0)),
            pl.BlockSpec((1, dp), lambda i: (0, 0)),
        ],
        out_specs=pl.BlockSpec((nbn, dp), lambda i: (i, 0)),
        out_shape=jax.ShapeDtypeStruct((n, dp), F32),
    )(n_feat_p, W0_p, b0_p)

    gather_k = _make_gather(n, e, dp, ep)
    scatter_k = _make_scatter(n, e, dp)

    bmm = pl.pallas_call(
        functools.partial(_bmm_body, d=d, dp=dp),
        grid=(nbe,),
        in_specs=[
            pl.BlockSpec((eb, dp), lambda i: (i, 0)),
            pl.BlockSpec((eb, dp), lambda i: (i, 0)),
            pl.BlockSpec(((d + 1) * dp, dp), lambda i: (0, 0)),
        ],
        out_specs=pl.BlockSpec((eb, dp), lambda i: (i, 0)),
        out_shape=jax.ShapeDtypeStruct((ep, dp), F32),
    )

    upd_specs = [
        pl.BlockSpec((_NC, nbn, dp), lambda i: (0, i, 0)),
        pl.BlockSpec((nbn, dp), lambda i: (i, 0)),
        pl.BlockSpec((dp, dp), lambda i: (0, 0)),
        pl.BlockSpec((dp, dp), lambda i: (0, 0)),
        pl.BlockSpec((1, dp), lambda i: (0, 0)),
        pl.BlockSpec((1, dp), lambda i: (0, 0)),
    ]
    upd = pl.pallas_call(
        _upd_body,
        grid=(n // nbn,),
        in_specs=upd_specs,
        out_specs=pl.BlockSpec((nbn, dp), lambda i: (i, 0)),
        out_shape=jax.ShapeDtypeStruct((n, dp), F32),
    )
    upd_final = pl.pallas_call(
        _upd_final_body,
        grid=(n // nbn,),
        in_specs=upd_specs + [pl.BlockSpec((nbn, dp), lambda i: (i, 0))],
        out_specs=pl.BlockSpec((nbn, dp), lambda i: (i, 0)),
        out_shape=jax.ShapeDtypeStruct((n, dp), F32),
    )

    out = out0
    for t in range(steps):
        h_src = gather_k(out, src2)
        msg = bmm(gmat, h_src, We2M)
        agg2 = scatter_k(msg, dst2, zero_nd)
        if t < steps - 1:
            out = upd(agg2, out, Wm1, Wm2, cb, bmp)
        else:
            out = upd_final(agg2, out, Wm1, Wm2, cb, bmp, n_feat_p)
    return out[:, :d]
